# baseline (speedup 1.0000x reference)
import jax
import jax.numpy as jnp
from jax import lax
from jax.experimental import pallas as pl
from jax.experimental.pallas import tpu as pltpu
from jax.experimental.pallas import tpu_sc as plsc

B = 16384
D = 32
NU = 1_000_000
NC, NS, L = 2, 16, 16
NW = NC * NS
BPW = B // NW
NG = BPW // L


def _mf_body(u_hbm, i_hbm, ub_hbm, uv_hbm, ib_hbm, iv_hbm, g_hbm, out_hbm,
             u_idx, i_idx, vu, vi, bu, bi, outv, gv, sem):
    wid = lax.axis_index("s") * NC + lax.axis_index("c")

    pltpu.sync_copy(u_hbm.at[wid], u_idx)
    pltpu.sync_copy(i_hbm.at[wid], i_idx)
    pltpu.sync_copy(g_hbm, gv)

    copies = []
    copies.append(pltpu.async_copy(ub_hbm.at[u_idx], bu, sem))
    copies.append(pltpu.async_copy(ib_hbm.at[i_idx], bi, sem))

    for d in range(D):
        copies.append(pltpu.async_copy(uv_hbm.at[d].at[u_idx], vu.at[d], sem))
        copies.append(pltpu.async_copy(iv_hbm.at[d].at[i_idx], vi.at[d], sem))
    for cp in copies:
        cp.wait()

    glob = gv[...]

    def group(gg, _):
        base = pl.multiple_of(gg * L, L)
        s = pl.ds(base, L)
        acc = bu[s] + bi[s] + glob
        for d in range(D):
            acc = acc + vu[d, s] * vi[d, s]
        outv[s] = acc
        return _

    lax.fori_loop(0, NG, group, 0)

    pltpu.sync_copy(outv, out_hbm.at[pl.ds(wid * BPW, BPW)])


@jax.jit
def _mf(u, i, user_bias, user_vec, item_bias, item_vec, glob_bias):
    mesh = plsc.VectorSubcoreMesh(core_axis_name="c", subcore_axis_name="s",
                                  num_cores=NC, num_subcores=NS)
    return pl.kernel(
        _mf_body,
        out_type=jax.ShapeDtypeStruct((B,), jnp.float32),
        mesh=mesh,
        compiler_params=pltpu.CompilerParams(
            needs_layout_passes=False, use_tc_tiling_on_sc=False),
        scratch_types=[
            pltpu.VMEM((BPW,), jnp.int32),
            pltpu.VMEM((BPW,), jnp.int32),
            pltpu.VMEM((D, BPW), jnp.float32),
            pltpu.VMEM((D, BPW), jnp.float32),
            pltpu.VMEM((BPW,), jnp.float32),
            pltpu.VMEM((BPW,), jnp.float32),
            pltpu.VMEM((BPW,), jnp.float32),
            pltpu.VMEM((L,), jnp.float32),
            pltpu.SemaphoreType.DMA,
        ],
    )(u, i, user_bias, user_vec.T, item_bias, item_vec.T, glob_bias)


def kernel(u, i, user_bias, user_vec, item_bias, item_vec, glob_bias):
    u = u.astype(jnp.int32).reshape(NW, BPW)
    i = i.astype(jnp.int32).reshape(NW, BPW)
    glob = jnp.broadcast_to(glob_bias.reshape(1), (L,))
    return _mf(u, i, user_bias, user_vec, item_bias, item_vec, glob)


# speedup vs baseline: 5.7016x; 5.7016x over previous
import jax
import jax.numpy as jnp
from jax import lax
from jax.experimental import pallas as pl
from jax.experimental.pallas import tpu as pltpu
from jax.experimental.pallas import tpu_sc as plsc

B = 16384
D = 32
NC, NS, L = 2, 16, 16
NW = NC * NS
BPW = B // NW
NG = BPW // L


def _mf_body(u_hbm, i_hbm, ub_hbm, uv_hbm, ib_hbm, iv_hbm, g_hbm, lane_hbm,
             out_hbm,
             u_idx, i_idx, vu, vi, bu, bi, outv, gv, lanev, tbuf, sem):
    wid = lax.axis_index("s") * NC + lax.axis_index("c")

    pltpu.sync_copy(u_hbm.at[wid], u_idx)
    pltpu.sync_copy(i_hbm.at[wid], i_idx)
    pltpu.sync_copy(g_hbm, gv)
    pltpu.sync_copy(lane_hbm, lanev)

    copies = [
        pltpu.async_copy(uv_hbm.at[u_idx], vu, sem),
        pltpu.async_copy(iv_hbm.at[i_idx], vi, sem),
        pltpu.async_copy(ub_hbm.at[u_idx], bu, sem),
        pltpu.async_copy(ib_hbm.at[i_idx], bi, sem),
    ]
    for cp in copies:
        cp.wait()

    gvv = gv[...]
    lane16 = lanev[...]

    def group(gg, _):
        base = pl.multiple_of(gg * L, L)
        for r in range(L):
            b = base + r
            a0 = vu[b, pl.ds(0, L)]
            a1 = vu[b, pl.ds(L, L)]
            c0 = vi[b, pl.ds(0, L)]
            c1 = vi[b, pl.ds(L, L)]
            plsc.store_scatter(tbuf, [lane16 + r], a0 * c0 + a1 * c1)
        s = pl.ds(base, L)
        acc = gvv + bu[s] + bi[s]
        for l in range(L):
            acc = acc + tbuf[pl.ds(l * L, L)]
        outv[s] = acc
        return _

    lax.fori_loop(0, NG, group, 0)

    pltpu.sync_copy(outv, out_hbm.at[pl.ds(wid * BPW, BPW)])


@jax.jit
def _mf(u, i, user_bias, user_vec, item_bias, item_vec, glob_bias, lane):
    mesh = plsc.VectorSubcoreMesh(core_axis_name="c", subcore_axis_name="s",
                                  num_cores=NC, num_subcores=NS)
    return pl.kernel(
        _mf_body,
        out_type=jax.ShapeDtypeStruct((B,), jnp.float32),
        mesh=mesh,
        compiler_params=pltpu.CompilerParams(
            needs_layout_passes=False, use_tc_tiling_on_sc=False),
        scratch_types=[
            pltpu.VMEM((BPW,), jnp.int32),
            pltpu.VMEM((BPW,), jnp.int32),
            pltpu.VMEM((BPW, D), jnp.float32),
            pltpu.VMEM((BPW, D), jnp.float32),
            pltpu.VMEM((BPW,), jnp.float32),
            pltpu.VMEM((BPW,), jnp.float32),
            pltpu.VMEM((BPW,), jnp.float32),
            pltpu.VMEM((L,), jnp.float32),
            pltpu.VMEM((L,), jnp.int32),
            pltpu.VMEM((L * L,), jnp.float32),
            pltpu.SemaphoreType.DMA,
        ],
    )(u, i, user_bias, user_vec, item_bias, item_vec, glob_bias, lane)


def kernel(u, i, user_bias, user_vec, item_bias, item_vec, glob_bias):
    u = u.astype(jnp.int32).reshape(NW, BPW)
    i = i.astype(jnp.int32).reshape(NW, BPW)
    glob = jnp.broadcast_to(glob_bias.reshape(1), (L,))
    lane = (jnp.arange(L, dtype=jnp.int32) * L)
    return _mf(u, i, user_bias, user_vec, item_bias, item_vec, glob, lane)
